# baseline (speedup 1.0000x reference)
import jax
import jax.numpy as jnp
from jax.experimental import pallas as pl

_NUM_EXPERTS = 64
_TOPK = 2
_RANK = 8
_ALPHA = 16.0
_SCALE = _ALPHA / float(_RANK)


def _fused_body(h_ref, wt_ref, aq_ref, av_ref, bq_ref, bv_ref, qo_ref, vo_ref):
    h = h_ref[...]
    f32 = jnp.float32
    logits = jnp.dot(h, wt_ref[...], preferred_element_type=f32)
    col = jax.lax.broadcasted_iota(jnp.int32, logits.shape, 1)
    big = jnp.int32(2 ** 30)
    m1 = jnp.max(logits, axis=-1, keepdims=True)
    i1 = jnp.min(jnp.where(logits == m1, col, big), axis=-1, keepdims=True)
    masked = jnp.where(col == i1, jnp.finfo(f32).min, logits)
    m2 = jnp.max(masked, axis=-1, keepdims=True)
    i2 = jnp.min(jnp.where(masked == m2, col, big), axis=-1, keepdims=True)
    e21 = jnp.exp(m2 - m1)
    denom = 1.0 + e21
    g1 = _SCALE / denom
    g2 = _SCALE * e21 / denom
    t = h.shape[0]
    ecol = jax.lax.broadcasted_iota(jnp.int32, (t, _NUM_EXPERTS * _RANK), 1) // _RANK
    gexp = jnp.where(ecol == i1, g1, 0.0) + jnp.where(ecol == i2, g2, 0.0)

    hb = h.astype(jnp.bfloat16)
    low_q = jnp.dot(hb, aq_ref[...], preferred_element_type=f32)
    glow_q = (low_q * gexp).astype(jnp.bfloat16)
    qo_ref[...] = jnp.dot(glow_q, bq_ref[...], preferred_element_type=f32)
    low_v = jnp.dot(hb, av_ref[...], preferred_element_type=f32)
    glow_v = (low_v * gexp).astype(jnp.bfloat16)
    vo_ref[...] = jnp.dot(glow_v, bv_ref[...], preferred_element_type=f32)


def kernel(hidden_states, router_weight, q_lora_a, q_lora_b, v_lora_a, v_lora_b):
    orig_shape = hidden_states.shape[:-1]
    d_model = hidden_states.shape[-1]
    h = hidden_states.reshape(-1, d_model)
    n = h.shape[0]
    e, _, r = q_lora_a.shape
    q_out = q_lora_b.shape[-1]
    v_out = v_lora_b.shape[-1]

    wt = router_weight.T
    bf16 = jnp.bfloat16
    aq = q_lora_a.transpose(1, 0, 2).reshape(d_model, e * r).astype(bf16)
    av = v_lora_a.transpose(1, 0, 2).reshape(d_model, e * r).astype(bf16)
    bq = q_lora_b.reshape(e * r, q_out).astype(bf16)
    bv = v_lora_b.reshape(e * r, v_out).astype(bf16)

    tile = 256
    grid = (n // tile,)
    const_spec = lambda shape: pl.BlockSpec(shape, lambda i: (0, 0))
    qo, vo = pl.pallas_call(
        _fused_body,
        grid=grid,
        in_specs=[
            pl.BlockSpec((tile, d_model), lambda i: (i, 0)),
            const_spec((d_model, e)),
            const_spec((d_model, e * r)),
            const_spec((d_model, e * r)),
            const_spec((e * r, q_out)),
            const_spec((e * r, v_out)),
        ],
        out_specs=[
            pl.BlockSpec((tile, q_out), lambda i: (i, 0)),
            pl.BlockSpec((tile, v_out), lambda i: (i, 0)),
        ],
        out_shape=[
            jax.ShapeDtypeStruct((n, q_out), jnp.float32),
            jax.ShapeDtypeStruct((n, v_out), jnp.float32),
        ],
    )(h, wt, aq, av, bq, bv)
    return (qo.reshape(orig_shape + (q_out,)), vo.reshape(orig_shape + (v_out,)))


# speedup vs baseline: 1.1221x; 1.1221x over previous
import jax
import jax.numpy as jnp
from jax.experimental import pallas as pl

_NUM_EXPERTS = 64
_TOPK = 2
_RANK = 8
_ALPHA = 16.0
_SCALE = _ALPHA / float(_RANK)


def _fused_body(h_ref, w_ref, aq_ref, av_ref, bq_ref, bv_ref, qo_ref, vo_ref):
    h = h_ref[...]
    f32 = jnp.float32
    logits = jax.lax.dot_general(
        h, w_ref[...], (((1,), (1,)), ((), ())), preferred_element_type=f32)
    col = jax.lax.broadcasted_iota(jnp.int32, logits.shape, 1)
    big = jnp.int32(2 ** 30)
    m1 = jnp.max(logits, axis=-1, keepdims=True)
    i1 = jnp.min(jnp.where(logits == m1, col, big), axis=-1, keepdims=True)
    masked = jnp.where(col == i1, jnp.finfo(f32).min, logits)
    m2 = jnp.max(masked, axis=-1, keepdims=True)
    i2 = jnp.min(jnp.where(masked == m2, col, big), axis=-1, keepdims=True)
    e21 = jnp.exp(m2 - m1)
    denom = 1.0 + e21
    g1 = _SCALE / denom
    g2 = _SCALE * e21 / denom
    t = h.shape[0]
    ecol = jax.lax.broadcasted_iota(jnp.int32, (t, _NUM_EXPERTS * _RANK), 1) // _RANK
    gexp = jnp.where(ecol == i1, g1, 0.0) + jnp.where(ecol == i2, g2, 0.0)

    bf = jnp.bfloat16
    hb = h.astype(bf)
    low_q = jnp.dot(hb, aq_ref[...].astype(bf), preferred_element_type=f32)
    glow_q = (low_q * gexp).astype(bf)
    qo_ref[...] = jnp.dot(glow_q, bq_ref[...].astype(bf), preferred_element_type=f32)
    low_v = jnp.dot(hb, av_ref[...].astype(bf), preferred_element_type=f32)
    glow_v = (low_v * gexp).astype(bf)
    vo_ref[...] = jnp.dot(glow_v, bv_ref[...].astype(bf), preferred_element_type=f32)


def kernel(hidden_states, router_weight, q_lora_a, q_lora_b, v_lora_a, v_lora_b):
    orig_shape = hidden_states.shape[:-1]
    d_model = hidden_states.shape[-1]
    h = hidden_states.reshape(-1, d_model)
    n = h.shape[0]
    e, _, r = q_lora_a.shape
    q_out = q_lora_b.shape[-1]
    v_out = v_lora_b.shape[-1]

    aq = q_lora_a.transpose(1, 0, 2).reshape(d_model, e * r)
    av = v_lora_a.transpose(1, 0, 2).reshape(d_model, e * r)
    bq = q_lora_b.reshape(e * r, q_out)
    bv = v_lora_b.reshape(e * r, v_out)

    tile = 256
    grid = (n // tile,)
    const_spec = lambda shape: pl.BlockSpec(shape, lambda i: (0, 0))
    qo, vo = pl.pallas_call(
        _fused_body,
        grid=grid,
        in_specs=[
            pl.BlockSpec((tile, d_model), lambda i: (i, 0)),
            const_spec((e, d_model)),
            const_spec((d_model, e * r)),
            const_spec((d_model, e * r)),
            const_spec((e * r, q_out)),
            const_spec((e * r, v_out)),
        ],
        out_specs=[
            pl.BlockSpec((tile, q_out), lambda i: (i, 0)),
            pl.BlockSpec((tile, v_out), lambda i: (i, 0)),
        ],
        out_shape=[
            jax.ShapeDtypeStruct((n, q_out), jnp.float32),
            jax.ShapeDtypeStruct((n, v_out), jnp.float32),
        ],
    )(h, router_weight, aq, av, bq, bv)
    return (qo.reshape(orig_shape + (q_out,)), vo.reshape(orig_shape + (v_out,)))


# speedup vs baseline: 1.2670x; 1.1291x over previous
import jax
import jax.numpy as jnp
from jax.experimental import pallas as pl

_NUM_EXPERTS = 64
_TOPK = 2
_RANK = 8
_ALPHA = 16.0
_SCALE = _ALPHA / float(_RANK)


def _fused_body(h_ref, w_ref, aq_ref, av_ref, bq_ref, bv_ref, qo_ref, vo_ref):
    h = h_ref[...]
    f32 = jnp.float32
    logits = jax.lax.dot_general(
        h, w_ref[...], (((1,), (1,)), ((), ())), preferred_element_type=f32)
    col = jax.lax.broadcasted_iota(jnp.int32, logits.shape, 1)
    big = jnp.int32(2 ** 30)
    m1 = jnp.max(logits, axis=-1, keepdims=True)
    i1 = jnp.min(jnp.where(logits == m1, col, big), axis=-1, keepdims=True)
    masked = jnp.where(col == i1, jnp.finfo(f32).min, logits)
    m2 = jnp.max(masked, axis=-1, keepdims=True)
    i2 = jnp.min(jnp.where(masked == m2, col, big), axis=-1, keepdims=True)
    e21 = jnp.exp(m2 - m1)
    denom = 1.0 + e21
    g1 = _SCALE / denom
    g2 = _SCALE * e21 / denom
    t = h.shape[0]
    ecol = jax.lax.broadcasted_iota(jnp.int32, (t, _NUM_EXPERTS * _RANK), 1) // _RANK
    gexp = jnp.where(ecol == i1, g1, 0.0) + jnp.where(ecol == i2, g2, 0.0)

    bf = jnp.bfloat16
    hb = h.astype(bf)
    low_q = jnp.dot(hb, aq_ref[...].astype(bf), preferred_element_type=f32)
    glow_q = (low_q * gexp).astype(bf)
    qo_ref[...] = jnp.dot(glow_q, bq_ref[...].astype(bf), preferred_element_type=f32)
    low_v = jnp.dot(hb, av_ref[...].astype(bf), preferred_element_type=f32)
    glow_v = (low_v * gexp).astype(bf)
    vo_ref[...] = jnp.dot(glow_v, bv_ref[...].astype(bf), preferred_element_type=f32)


def kernel(hidden_states, router_weight, q_lora_a, q_lora_b, v_lora_a, v_lora_b):
    orig_shape = hidden_states.shape[:-1]
    d_model = hidden_states.shape[-1]
    h = hidden_states.reshape(-1, d_model)
    n = h.shape[0]
    e, _, r = q_lora_a.shape
    q_out = q_lora_b.shape[-1]
    v_out = v_lora_b.shape[-1]

    aq = q_lora_a.transpose(1, 0, 2).reshape(d_model, e * r)
    av = v_lora_a.transpose(1, 0, 2).reshape(d_model, e * r)
    bq = q_lora_b.reshape(e * r, q_out)
    bv = v_lora_b.reshape(e * r, v_out)

    tile = 512
    grid = (n // tile,)
    const_spec = lambda shape: pl.BlockSpec(shape, lambda i: (0, 0))
    qo, vo = pl.pallas_call(
        _fused_body,
        grid=grid,
        in_specs=[
            pl.BlockSpec((tile, d_model), lambda i: (i, 0)),
            const_spec((e, d_model)),
            const_spec((d_model, e * r)),
            const_spec((d_model, e * r)),
            const_spec((e * r, q_out)),
            const_spec((e * r, v_out)),
        ],
        out_specs=[
            pl.BlockSpec((tile, q_out), lambda i: (i, 0)),
            pl.BlockSpec((tile, v_out), lambda i: (i, 0)),
        ],
        out_shape=[
            jax.ShapeDtypeStruct((n, q_out), jnp.float32),
            jax.ShapeDtypeStruct((n, v_out), jnp.float32),
        ],
    )(h, router_weight, aq, av, bq, bv)
    return (qo.reshape(orig_shape + (q_out,)), vo.reshape(orig_shape + (v_out,)))


# speedup vs baseline: 1.2806x; 1.0107x over previous
import jax
import jax.numpy as jnp
from jax.experimental import pallas as pl

_NUM_EXPERTS = 64
_TOPK = 2
_RANK = 8
_ALPHA = 16.0
_SCALE = _ALPHA / float(_RANK)


def _fused_body(h_ref, w_ref, aq_ref, av_ref, bq_ref, bv_ref, qo_ref, vo_ref):
    h = h_ref[...]
    f32 = jnp.float32
    logits = jax.lax.dot_general(
        h, w_ref[...], (((1,), (1,)), ((), ())), preferred_element_type=f32)
    col = jax.lax.broadcasted_iota(jnp.int32, logits.shape, 1)
    big = jnp.int32(2 ** 30)
    m1 = jnp.max(logits, axis=-1, keepdims=True)
    i1 = jnp.min(jnp.where(logits == m1, col, big), axis=-1, keepdims=True)
    masked = jnp.where(col == i1, jnp.finfo(f32).min, logits)
    m2 = jnp.max(masked, axis=-1, keepdims=True)
    i2 = jnp.min(jnp.where(masked == m2, col, big), axis=-1, keepdims=True)
    e21 = jnp.exp(m2 - m1)
    denom = 1.0 + e21
    g1 = _SCALE / denom
    g2 = _SCALE * e21 / denom
    t = h.shape[0]
    ecol = jax.lax.broadcasted_iota(jnp.int32, (t, _NUM_EXPERTS * _RANK), 1) // _RANK
    gexp = jnp.where(ecol == i1, g1, 0.0) + jnp.where(ecol == i2, g2, 0.0)

    bf = jnp.bfloat16
    hb = h.astype(bf)
    low_q = jnp.dot(hb, aq_ref[...].astype(bf), preferred_element_type=f32)
    glow_q = (low_q * gexp).astype(bf)
    qo_ref[...] = jnp.dot(glow_q, bq_ref[...].astype(bf), preferred_element_type=f32)
    low_v = jnp.dot(hb, av_ref[...].astype(bf), preferred_element_type=f32)
    glow_v = (low_v * gexp).astype(bf)
    vo_ref[...] = jnp.dot(glow_v, bv_ref[...].astype(bf), preferred_element_type=f32)


def kernel(hidden_states, router_weight, q_lora_a, q_lora_b, v_lora_a, v_lora_b):
    orig_shape = hidden_states.shape[:-1]
    d_model = hidden_states.shape[-1]
    h = hidden_states.reshape(-1, d_model)
    n = h.shape[0]
    e, _, r = q_lora_a.shape
    q_out = q_lora_b.shape[-1]
    v_out = v_lora_b.shape[-1]

    aq = q_lora_a.transpose(1, 0, 2).reshape(d_model, e * r)
    av = v_lora_a.transpose(1, 0, 2).reshape(d_model, e * r)
    bq = q_lora_b.reshape(e * r, q_out)
    bv = v_lora_b.reshape(e * r, v_out)

    tile = 1024
    grid = (n // tile,)
    const_spec = lambda shape: pl.BlockSpec(shape, lambda i: (0, 0))
    qo, vo = pl.pallas_call(
        _fused_body,
        grid=grid,
        in_specs=[
            pl.BlockSpec((tile, d_model), lambda i: (i, 0)),
            const_spec((e, d_model)),
            const_spec((d_model, e * r)),
            const_spec((d_model, e * r)),
            const_spec((e * r, q_out)),
            const_spec((e * r, v_out)),
        ],
        out_specs=[
            pl.BlockSpec((tile, q_out), lambda i: (i, 0)),
            pl.BlockSpec((tile, v_out), lambda i: (i, 0)),
        ],
        out_shape=[
            jax.ShapeDtypeStruct((n, q_out), jnp.float32),
            jax.ShapeDtypeStruct((n, v_out), jnp.float32),
        ],
    )(h, router_weight, aq, av, bq, bv)
    return (qo.reshape(orig_shape + (q_out,)), vo.reshape(orig_shape + (v_out,)))
